# baseline (speedup 1.0000x reference)
import functools

import jax
import jax.numpy as jnp
from jax.experimental import pallas as pl
from jax.experimental.pallas import tpu as pltpu

_THRESHOLD = 0.5


def _gate_kernel(x_ref, wa_ref, m_ref, scale_ref, bias_ref,
                 gout_ref, gw_ref, xb_ref):
    xb = x_ref[...].astype(jnp.bfloat16)
    xb_ref[...] = xb
    gh = jax.lax.dot_general(
        xb, wa_ref[...].astype(jnp.bfloat16), (((1,), (1,)), ((), ())),
        preferred_element_type=jnp.float32)
    g2 = gh * gh
    s2 = jax.lax.dot_general(
        g2, m_ref[...], (((1,), (0,)), ((), ())),
        precision=jax.lax.Precision.HIGHEST,
        preferred_element_type=jnp.float32)
    scores = jnp.sqrt(s2 + 1e-6) * scale_ref[...] - bias_ref[...]
    mask = scores >= _THRESHOLD
    gout_ref[...] = jnp.where(mask, scores, -jnp.inf)
    gw_ref[...] = jnp.where(mask, scores, 0.0)


def _ffn_kernel(x_ref, gw_ref, wg_ref, wu_ref, wd_ref, out_ref, *, half, tb):
    e = pl.program_id(0)
    t = pl.program_id(1)

    @pl.when((t == 0) & (e % half == 0))
    def _init():
        out_ref[...] = jnp.zeros_like(out_ref)

    x = x_ref[...]
    wg = wg_ref[0]
    wu = wu_ref[0]
    wd = wd_ref[0]
    xg = jax.lax.dot_general(x, wg, (((1,), (0,)), ((), ())),
                             preferred_element_type=jnp.float32)
    xu = jax.lax.dot_general(x, wu, (((1,), (0,)), ((), ())),
                             preferred_element_type=jnp.float32)
    h = xg * jax.nn.sigmoid(xg) * xu
    gw = gw_ref[...]
    lane = jax.lax.broadcasted_iota(jnp.int32, gw.shape, 1)
    gcol = jnp.sum(jnp.where(lane == e, gw, 0.0), axis=1, keepdims=True)
    hs = (h * gcol).astype(jnp.bfloat16)
    contrib = jax.lax.dot_general(hs, wd, (((1,), (0,)), ((), ())),
                                  preferred_element_type=jnp.float32)
    out_ref[0, pl.ds(t * tb, tb), :] += contrib


def kernel(hidden_states, W_A, gate_scale, gate_bias, W_gate, W_up, W_down):
    orig_shape = hidden_states.shape
    D = orig_shape[-1]
    x = hidden_states.reshape(-1, D)
    N = x.shape[0]
    E, R, _ = W_A.shape
    DFF = W_gate.shape[1]

    wa2 = W_A.reshape(E * R, D)
    m = jnp.repeat(jnp.eye(E, dtype=jnp.float32), R, axis=0) / R
    TGB = 512
    gate_out, gw, xb = pl.pallas_call(
        _gate_kernel,
        grid=(N // TGB,),
        in_specs=[
            pl.BlockSpec((TGB, D), lambda t: (t, 0)),
            pl.BlockSpec((E * R, D), lambda t: (0, 0)),
            pl.BlockSpec((E * R, E), lambda t: (0, 0)),
            pl.BlockSpec((1, E), lambda t: (0, 0)),
            pl.BlockSpec((1, E), lambda t: (0, 0)),
        ],
        out_specs=[
            pl.BlockSpec((TGB, E), lambda t: (t, 0)),
            pl.BlockSpec((TGB, E), lambda t: (t, 0)),
            pl.BlockSpec((TGB, D), lambda t: (t, 0)),
        ],
        out_shape=[
            jax.ShapeDtypeStruct((N, E), jnp.float32),
            jax.ShapeDtypeStruct((N, E), jnp.float32),
            jax.ShapeDtypeStruct((N, D), jnp.bfloat16),
        ],
    )(x, wa2, m, gate_scale.reshape(1, E), gate_bias.reshape(1, E))

    wgt = W_gate.astype(jnp.bfloat16).transpose(0, 2, 1)
    wut = W_up.astype(jnp.bfloat16).transpose(0, 2, 1)
    wdt = W_down.astype(jnp.bfloat16).transpose(0, 2, 1)
    half = E // 2
    TB = 512
    T = N // TB
    out2 = pl.pallas_call(
        functools.partial(_ffn_kernel, half=half, tb=TB),
        grid=(E, T),
        in_specs=[
            pl.BlockSpec((TB, D), lambda e, t: (t, 0)),
            pl.BlockSpec((TB, E), lambda e, t: (t, 0)),
            pl.BlockSpec((1, D, DFF), lambda e, t: (e, 0, 0)),
            pl.BlockSpec((1, D, DFF), lambda e, t: (e, 0, 0)),
            pl.BlockSpec((1, DFF, D), lambda e, t: (e, 0, 0)),
        ],
        out_specs=pl.BlockSpec((1, N, D), lambda e, t: (e // half, 0, 0)),
        out_shape=jax.ShapeDtypeStruct((2, N, D), jnp.float32),
        compiler_params=pltpu.CompilerParams(
            dimension_semantics=("parallel", "arbitrary"),
            vmem_limit_bytes=64 * 1024 * 1024),
    )(xb, gw, wgt, wut, wdt)
    out = (out2[0] + out2[1]).reshape(orig_shape)
    return out, gate_out.reshape(orig_shape[:-1] + (E,))


# speedup vs baseline: 1.0545x; 1.0545x over previous
import functools

import jax
import jax.numpy as jnp
from jax.experimental import pallas as pl
from jax.experimental.pallas import tpu as pltpu

_THRESHOLD = 0.5


def _gate_kernel(x_ref, wa_ref, m_ref, scale_ref, bias_ref,
                 gout_ref, gw_ref, xb_ref):
    xb = x_ref[...].astype(jnp.bfloat16)
    xb_ref[...] = xb
    gh = jax.lax.dot_general(
        xb, wa_ref[...].astype(jnp.bfloat16), (((1,), (1,)), ((), ())),
        preferred_element_type=jnp.float32)
    g2 = gh * gh
    s2 = jax.lax.dot_general(
        g2, m_ref[...], (((1,), (0,)), ((), ())),
        precision=jax.lax.Precision.HIGHEST,
        preferred_element_type=jnp.float32)
    scores = jnp.sqrt(s2 + 1e-6) * scale_ref[...] - bias_ref[...]
    mask = scores >= _THRESHOLD
    gout_ref[...] = jnp.where(mask, scores, -jnp.inf)
    gw_ref[...] = jnp.where(mask, scores, 0.0)


def _ffn_kernel(x_ref, gw_ref, wg_ref, wu_ref, wd_ref, out_ref,
                wgt_ref, wut_ref, wdt_ref, *, half, tb):
    e = pl.program_id(0)
    t = pl.program_id(2)

    @pl.when((t == 0) & (e % half == 0) & (pl.program_id(1) == 0))
    def _init():
        out_ref[...] = jnp.zeros_like(out_ref)

    @pl.when(t == 0)
    def _prep_weights():
        wgt_ref[...] = wg_ref[0].astype(jnp.bfloat16).T
        wut_ref[...] = wu_ref[0].astype(jnp.bfloat16).T
        wdt_ref[...] = wd_ref[0].astype(jnp.bfloat16).T

    x = x_ref[pl.ds(t * tb, tb), :]
    xg = jax.lax.dot_general(x, wgt_ref[...], (((1,), (0,)), ((), ())),
                             preferred_element_type=jnp.float32)
    xu = jax.lax.dot_general(x, wut_ref[...], (((1,), (0,)), ((), ())),
                             preferred_element_type=jnp.float32)
    h = xg * jax.nn.sigmoid(xg) * xu
    gw = gw_ref[pl.ds(t * tb, tb), :]
    lane = jax.lax.broadcasted_iota(jnp.int32, gw.shape, 1)
    gcol = jnp.sum(jnp.where(lane == e, gw, 0.0), axis=1, keepdims=True)
    hs = (h * gcol).astype(jnp.bfloat16)
    contrib = jax.lax.dot_general(hs, wdt_ref[...], (((1,), (0,)), ((), ())),
                                  preferred_element_type=jnp.float32)
    out_ref[0, pl.ds(t * tb, tb), :] += contrib


def kernel(hidden_states, W_A, gate_scale, gate_bias, W_gate, W_up, W_down):
    orig_shape = hidden_states.shape
    D = orig_shape[-1]
    x = hidden_states.reshape(-1, D)
    N = x.shape[0]
    E, R, _ = W_A.shape
    DFF = W_gate.shape[1]

    wa2 = W_A.reshape(E * R, D)
    m = jnp.repeat(jnp.eye(E, dtype=jnp.float32), R, axis=0) / R
    TGB = 512
    gate_out, gw, xb = pl.pallas_call(
        _gate_kernel,
        grid=(N // TGB,),
        in_specs=[
            pl.BlockSpec((TGB, D), lambda t: (t, 0)),
            pl.BlockSpec((E * R, D), lambda t: (0, 0)),
            pl.BlockSpec((E * R, E), lambda t: (0, 0)),
            pl.BlockSpec((1, E), lambda t: (0, 0)),
            pl.BlockSpec((1, E), lambda t: (0, 0)),
        ],
        out_specs=[
            pl.BlockSpec((TGB, E), lambda t: (t, 0)),
            pl.BlockSpec((TGB, E), lambda t: (t, 0)),
            pl.BlockSpec((TGB, D), lambda t: (t, 0)),
        ],
        out_shape=[
            jax.ShapeDtypeStruct((N, E), jnp.float32),
            jax.ShapeDtypeStruct((N, E), jnp.float32),
            jax.ShapeDtypeStruct((N, D), jnp.bfloat16),
        ],
    )(x, wa2, m, gate_scale.reshape(1, E), gate_bias.reshape(1, E))

    half = E // 2
    FB = 768 if DFF % 768 == 0 else DFF
    F = DFF // FB
    TB = 512
    T = N // TB
    out2 = pl.pallas_call(
        functools.partial(_ffn_kernel, half=half, tb=TB),
        grid=(E, F, T),
        in_specs=[
            pl.BlockSpec((N, D), lambda e, f, t: (0, 0)),
            pl.BlockSpec((N, E), lambda e, f, t: (0, 0)),
            pl.BlockSpec((1, FB, D), lambda e, f, t: (e, f, 0)),
            pl.BlockSpec((1, FB, D), lambda e, f, t: (e, f, 0)),
            pl.BlockSpec((1, D, FB), lambda e, f, t: (e, 0, f)),
        ],
        out_specs=pl.BlockSpec((1, N, D), lambda e, f, t: (e // half, 0, 0)),
        out_shape=jax.ShapeDtypeStruct((2, N, D), jnp.float32),
        scratch_shapes=[
            pltpu.VMEM((D, FB), jnp.bfloat16),
            pltpu.VMEM((D, FB), jnp.bfloat16),
            pltpu.VMEM((FB, D), jnp.bfloat16),
        ],
        compiler_params=pltpu.CompilerParams(
            dimension_semantics=("parallel", "arbitrary", "arbitrary"),
            vmem_limit_bytes=64 * 1024 * 1024),
    )(xb, gw, W_gate, W_up, W_down)
    out = (out2[0] + out2[1]).reshape(orig_shape)
    return out, gate_out.reshape(orig_shape[:-1] + (E,))
